# Initial kernel scaffold; baseline (speedup 1.0000x reference)
import jax
import jax.numpy as jnp
from jax.experimental import pallas as pl


def kernel(batch_seqs, vectors):
    raise NotImplementedError("write your pallas kernel here")


# speedup vs baseline: 5.9913x; 5.9913x over previous
import functools

import jax
import jax.numpy as jnp
from jax import lax
from jax.experimental import pallas as pl
from jax.experimental.pallas import tpu as pltpu
from jax.experimental.pallas import tpu_sc as plsc

VOCAB = 100000
EMBED_DIM = 64
BATCH = 16384
HIST_LEN = 50
B_FLAT = BATCH * HIST_LEN

_NUM_CORES = 2
_NUM_SUBCORES = 16
_NW = _NUM_CORES * _NUM_SUBCORES
_B_PER_W = B_FLAT // _NW
_IDX_CHUNK = 128
_GATHERS_PER_STEP = 4
_CHUNK = _IDX_CHUNK * _GATHERS_PER_STEP
_STEPS = _B_PER_W // _CHUNK


@functools.partial(
    pl.kernel,
    mesh=plsc.VectorSubcoreMesh(core_axis_name="c", subcore_axis_name="s"),
    out_type=jax.ShapeDtypeStruct((B_FLAT, EMBED_DIM), jnp.float32),
    scratch_types=[
        pltpu.VMEM((_B_PER_W,), jnp.int32),
        pltpu.VMEM((_CHUNK, EMBED_DIM), jnp.float32),
        pltpu.SemaphoreType.DMA,
    ],
    compiler_params=pltpu.CompilerParams(use_tc_tiling_on_sc=False),
)
def _gather_kernel(seq_hbm, table_hbm, out_hbm, idx_v, rows_v, sem):
    wid = lax.axis_index("s") * _NUM_CORES + lax.axis_index("c")
    base = wid * _B_PER_W
    pltpu.sync_copy(seq_hbm.at[pl.ds(base, _B_PER_W)], idx_v)

    def step(j, carry):
        off = pl.multiple_of(j * _CHUNK, _CHUNK)
        copies = []
        for g in range(_GATHERS_PER_STEP):
            copies.append(pltpu.async_copy(
                table_hbm.at[idx_v.at[pl.ds(off + g * _IDX_CHUNK, _IDX_CHUNK)]],
                rows_v.at[pl.ds(g * _IDX_CHUNK, _IDX_CHUNK)],
                sem,
            ))
        for cp in copies:
            cp.wait()
        pltpu.sync_copy(rows_v, out_hbm.at[pl.ds(base + off, _CHUNK)])
        return carry

    lax.fori_loop(0, _STEPS, step, 0)


def kernel(batch_seqs, vectors):
    flat_idx = batch_seqs.reshape(B_FLAT)
    out = _gather_kernel(flat_idx, vectors)
    return out.reshape(BATCH, HIST_LEN, EMBED_DIM)


# speedup vs baseline: 6.2441x; 1.0422x over previous
import functools

import jax
import jax.numpy as jnp
from jax import lax
from jax.experimental import pallas as pl
from jax.experimental.pallas import tpu as pltpu
from jax.experimental.pallas import tpu_sc as plsc

VOCAB = 100000
EMBED_DIM = 64
BATCH = 16384
HIST_LEN = 50
B_FLAT = BATCH * HIST_LEN

_NUM_CORES = 2
_NUM_SUBCORES = 16
_NW = _NUM_CORES * _NUM_SUBCORES
_B_PER_W = B_FLAT // _NW
_IDX_CHUNK = 128
_GATHERS_PER_STEP = 4
_CHUNK = _IDX_CHUNK * _GATHERS_PER_STEP
_STEPS = _B_PER_W // _CHUNK


@functools.partial(
    pl.kernel,
    mesh=plsc.VectorSubcoreMesh(core_axis_name="c", subcore_axis_name="s"),
    out_type=jax.ShapeDtypeStruct((B_FLAT, EMBED_DIM), jnp.float32),
    scratch_types=[
        pltpu.VMEM((_B_PER_W,), jnp.int32),
        pltpu.VMEM((_CHUNK, EMBED_DIM), jnp.float32),
        pltpu.VMEM((_CHUNK, EMBED_DIM), jnp.float32),
        pltpu.SemaphoreType.DMA,
        pltpu.SemaphoreType.DMA,
        pltpu.SemaphoreType.DMA,
        pltpu.SemaphoreType.DMA,
    ],
    compiler_params=pltpu.CompilerParams(use_tc_tiling_on_sc=False),
)
def _gather_kernel(seq_hbm, table_hbm, out_hbm, idx_v,
                   rows0, rows1, sem_g0, sem_g1, sem_o0, sem_o1):
    wid = lax.axis_index("s") * _NUM_CORES + lax.axis_index("c")
    base = wid * _B_PER_W
    pltpu.sync_copy(seq_hbm.at[pl.ds(base, _B_PER_W)], idx_v)

    rows = (rows0, rows1)
    sem_g = (sem_g0, sem_g1)
    sem_o = (sem_o0, sem_o1)

    def fire_gathers(s, buf):
        off = pl.multiple_of(s * _CHUNK, _CHUNK)
        for g in range(_GATHERS_PER_STEP):
            pltpu.async_copy(
                table_hbm.at[idx_v.at[pl.ds(off + g * _IDX_CHUNK, _IDX_CHUNK)]],
                rows[buf].at[pl.ds(g * _IDX_CHUNK, _IDX_CHUNK)],
                sem_g[buf],
            )

    def wait_gathers(buf):
        pltpu.make_async_copy(
            table_hbm.at[pl.ds(0, _CHUNK)], rows[buf], sem_g[buf]).wait()

    def fire_out(s, buf):
        off = pl.multiple_of(s * _CHUNK, _CHUNK)
        pltpu.async_copy(rows[buf], out_hbm.at[pl.ds(base + off, _CHUNK)],
                         sem_o[buf])

    def wait_out(buf):
        pltpu.make_async_copy(rows[buf], out_hbm.at[pl.ds(base, _CHUNK)],
                              sem_o[buf]).wait()

    def steady(s, cur):
        nxt = 1 - cur
        wait_out(nxt)
        fire_gathers(s + 1, nxt)
        wait_gathers(cur)
        fire_out(s, cur)

    fire_gathers(0, 0)
    fire_gathers(1, 1)
    wait_gathers(0)
    fire_out(0, 0)

    def pair(k, carry):
        s = 2 * k + 1
        steady(s, 1)
        steady(s + 1, 0)
        return carry

    lax.fori_loop(0, (_STEPS - 2) // 2, pair, 0)

    wait_out(0)
    wait_gathers(1)
    fire_out(_STEPS - 1, 1)
    wait_out(1)


def kernel(batch_seqs, vectors):
    flat_idx = batch_seqs.reshape(B_FLAT)
    out = _gather_kernel(flat_idx, vectors)
    return out.reshape(BATCH, HIST_LEN, EMBED_DIM)
